# baseline (speedup 1.0000x reference)
import jax
import jax.numpy as jnp
from jax import lax
from jax.experimental import pallas as pl
from jax.experimental.pallas import tpu as pltpu
from jax.experimental.pallas import tpu_sc as plsc

N_NODES = 10000
N_EDGES = 320000
D = 128
N_HOPS = 3

NC = 2
NS = 16
L = 16
DH = D // NC

CHUNK = 128
NCH = 160
EPT = NCH * CHUNK
EP = NS * EPT
NP = 10112
RPT = NP // NS


def _hop_body(agg_hbm, row_hbm, col_hbm, tr_hbm, out_hbm,
              table, acc, row_v, col_v, tr_v, gat_v, sem_i, sem_g, sem_s):
    c = lax.axis_index("c")
    s = lax.axis_index("s")
    base = s * EPT

    def idx_start(ch, b):
        ebase = base + ch * CHUNK
        pltpu.async_copy(row_hbm.at[pl.ds(ebase, CHUNK)], row_v.at[b], sem_i.at[b])
        pltpu.async_copy(col_hbm.at[pl.ds(ebase, CHUNK)], col_v.at[b], sem_i.at[b])
        pltpu.async_copy(tr_hbm.at[pl.ds(ebase, CHUNK)], tr_v.at[b], sem_i.at[b])

    def idx_wait(ch, b):
        ebase = base + ch * CHUNK
        pltpu.make_async_copy(row_hbm.at[pl.ds(ebase, CHUNK)], row_v.at[b], sem_i.at[b]).wait()
        pltpu.make_async_copy(col_hbm.at[pl.ds(ebase, CHUNK)], col_v.at[b], sem_i.at[b]).wait()
        pltpu.make_async_copy(tr_hbm.at[pl.ds(ebase, CHUNK)], tr_v.at[b], sem_i.at[b]).wait()

    def gat_start(b, g):
        pltpu.async_copy(table.at[row_v.at[b]], gat_v.at[g], sem_g.at[g])

    def gat_wait(b, g):
        pltpu.make_async_copy(table.at[row_v.at[b]], gat_v.at[g], sem_g.at[g]).wait()

    def scale(b, gb):
        @pl.loop(0, CHUNK // L)
        def _(g):
            t16 = tr_v[b, pl.ds(g * L, L)]
            for l in range(L):
                e = g * L + l
                t = t16[l]
                for d in range(DH // L):
                    sl = pl.ds(d * L, L)
                    gat_v[gb, e, sl] = gat_v[gb, e, sl] * t

    def scat_start(b, gb):
        pltpu.async_copy(gat_v.at[gb], acc.at[col_v.at[b]], sem_s.at[gb], add=True)

    def scat_wait(gb):
        pltpu.make_async_copy(gat_v.at[gb], acc.at[col_v.at[0]], sem_s.at[gb]).wait()

    pltpu.sync_copy(agg_hbm.at[c, pl.ds(s * RPT, RPT)], table.at[pl.ds(s * RPT, RPT)])

    @pl.loop(0, CHUNK)
    def _(r):
        for k in range(DH // L):
            gat_v[0, r, pl.ds(k * L, L)] = jnp.zeros((L,), jnp.float32)

    nz = RPT // CHUNK
    rem = RPT - nz * CHUNK
    for j in range(nz):
        pltpu.sync_copy(gat_v.at[0], acc.at[pl.ds(s * RPT + j * CHUNK, CHUNK)])
    pltpu.sync_copy(gat_v.at[0, pl.ds(0, rem)],
                    acc.at[pl.ds(s * RPT + nz * CHUNK, rem)])
    plsc.subcore_barrier()

    for p in range(3):
        idx_start(p, p)
    idx_wait(0, 0)
    gat_start(0, 0)

    @pl.loop(0, NCH)
    def _(ch):
        ib = lax.rem(ch, 4)
        gb = lax.rem(ch, 2)

        @pl.when(ch + 1 < NCH)
        def _():
            idx_wait(ch + 1, lax.rem(ch + 1, 4))

        @pl.when(jnp.logical_and(ch >= 1, ch + 1 < NCH))
        def _():
            scat_wait(1 - gb)

        @pl.when(ch + 1 < NCH)
        def _():
            gat_start(lax.rem(ch + 1, 4), 1 - gb)

        gat_wait(ib, gb)
        scale(ib, gb)
        scat_start(ib, gb)

        @pl.when(ch + 3 < NCH)
        def _():
            idx_start(ch + 3, lax.rem(ch + 3, 4))

    scat_wait(0)
    scat_wait(1)
    plsc.subcore_barrier()
    for j in range(nz):
        rs = s * RPT + j * CHUNK
        pltpu.sync_copy(acc.at[pl.ds(rs, CHUNK)], out_hbm.at[c, pl.ds(rs, CHUNK)])
    rs = s * RPT + nz * CHUNK
    pltpu.sync_copy(acc.at[pl.ds(rs, rem)], out_hbm.at[c, pl.ds(rs, rem)])


_hop = pl.kernel(
    _hop_body,
    out_type=jax.ShapeDtypeStruct((NC, NP, DH), jnp.float32),
    mesh=plsc.VectorSubcoreMesh(core_axis_name="c", subcore_axis_name="s"),
    scratch_types=[
        pltpu.VMEM_SHARED((NP, DH), jnp.float32),
        pltpu.VMEM_SHARED((NP, DH), jnp.float32),
        pltpu.VMEM((4, CHUNK), jnp.int32),
        pltpu.VMEM((4, CHUNK), jnp.int32),
        pltpu.VMEM((4, CHUNK), jnp.float32),
        pltpu.VMEM((2, CHUNK, DH), jnp.float32),
        pltpu.SemaphoreType.DMA((4,)),
        pltpu.SemaphoreType.DMA((2,)),
        pltpu.SemaphoreType.DMA((2,)),
    ],
)


@jax.jit
def kernel(embed, edge_index, trend):
    row = edge_index[0].astype(jnp.int32)
    col = edge_index[1].astype(jnp.int32)
    pad = EP - N_EDGES
    row = jnp.concatenate([row, jnp.zeros((pad,), jnp.int32)])
    col = jnp.concatenate([col, jnp.zeros((pad,), jnp.int32)])
    tr = jnp.concatenate([trend, jnp.zeros((pad,), jnp.float32)])

    npad = NP - N_NODES
    agg2 = jnp.stack([
        jnp.concatenate([embed[:, :DH], jnp.zeros((npad, DH), jnp.float32)]),
        jnp.concatenate([embed[:, DH:], jnp.zeros((npad, DH), jnp.float32)]),
    ])
    embs = [embed]
    for _ in range(N_HOPS):
        agg2 = _hop(agg2, row, col, tr)
        embs.append(jnp.concatenate([agg2[0, :N_NODES], agg2[1, :N_NODES]], axis=1))
    return jnp.stack(embs, axis=1)


# speedup vs baseline: 1.0590x; 1.0590x over previous
import jax
import jax.numpy as jnp
from jax import lax
from jax.experimental import pallas as pl
from jax.experimental.pallas import tpu as pltpu
from jax.experimental.pallas import tpu_sc as plsc

N_NODES = 10000
N_EDGES = 320000
D = 128
N_HOPS = 3

NC = 2
NS = 16
L = 16
DH = D // NC

CHUNK = 128
NCH = 160
EPT = NCH * CHUNK
EP = NS * EPT
NP = 10112
RPT = NP // NS


def _hop_body(agg_hbm, row_hbm, col_hbm, tr_hbm, out_hbm,
              tabA, tabB, row_v, col_v, tr_v, gat_v, sem_i, sem_g):
    c = lax.axis_index("c")
    s = lax.axis_index("s")
    base = s * EPT

    def idx_start(ch, b):
        ebase = base + ch * CHUNK
        pltpu.async_copy(row_hbm.at[pl.ds(ebase, CHUNK)], row_v.at[b], sem_i.at[b])
        pltpu.async_copy(col_hbm.at[pl.ds(ebase, CHUNK)], col_v.at[b], sem_i.at[b])
        pltpu.async_copy(tr_hbm.at[pl.ds(ebase, CHUNK)], tr_v.at[b], sem_i.at[b])

    def idx_wait(ch, b):
        ebase = base + ch * CHUNK
        pltpu.make_async_copy(row_hbm.at[pl.ds(ebase, CHUNK)], row_v.at[b], sem_i.at[b]).wait()
        pltpu.make_async_copy(col_hbm.at[pl.ds(ebase, CHUNK)], col_v.at[b], sem_i.at[b]).wait()
        pltpu.make_async_copy(tr_hbm.at[pl.ds(ebase, CHUNK)], tr_v.at[b], sem_i.at[b]).wait()

    def zero_gat0():
        @pl.loop(0, CHUNK)
        def _(r):
            for k in range(DH // L):
                gat_v[0, r, pl.ds(k * L, L)] = jnp.zeros((L,), jnp.float32)

    nz = RPT // CHUNK
    rem = RPT - nz * CHUNK

    def zero_acc(acc):
        zero_gat0()
        for j in range(nz):
            pltpu.sync_copy(gat_v.at[0], acc.at[pl.ds(s * RPT + j * CHUNK, CHUNK)])
        pltpu.sync_copy(gat_v.at[0, pl.ds(0, rem)],
                        acc.at[pl.ds(s * RPT + nz * CHUNK, rem)])

    def run_hop(table, acc):
        def gat_start(b, g):
            pltpu.async_copy(table.at[row_v.at[b]], gat_v.at[g], sem_g.at[g])

        def gat_wait(b, g):
            pltpu.make_async_copy(table.at[row_v.at[b]], gat_v.at[g], sem_g.at[g]).wait()

        def scale_scatter(b, gb):
            @pl.loop(0, CHUNK // L)
            def _(g):
                t16 = tr_v[b, pl.ds(g * L, L)]
                for l in range(L):
                    e = g * L + l
                    t = t16[l]
                    for d in range(DH // L):
                        sl = pl.ds(d * L, L)
                        gat_v[gb, e, sl] = gat_v[gb, e, sl] * t

            pltpu.sync_copy(gat_v.at[gb], acc.at[col_v.at[b]], add=True)

        for p in range(4):
            idx_start(p, p)
        idx_wait(0, 0)
        gat_start(0, 0)

        @pl.loop(0, NCH)
        def _(ch):
            ib = lax.rem(ch, 4)
            gb = lax.rem(ch, 2)

            @pl.when(ch + 1 < NCH)
            def _():
                idx_wait(ch + 1, lax.rem(ch + 1, 4))
                gat_start(lax.rem(ch + 1, 4), 1 - gb)

            gat_wait(ib, gb)
            scale_scatter(ib, gb)

            @pl.when(ch + 4 < NCH)
            def _():
                idx_start(ch + 4, ib)

    def flush(acc, h):
        for j in range(nz):
            rs = s * RPT + j * CHUNK
            pltpu.sync_copy(acc.at[pl.ds(rs, CHUNK)], out_hbm.at[h, c, pl.ds(rs, CHUNK)])
        rs = s * RPT + nz * CHUNK
        pltpu.sync_copy(acc.at[pl.ds(rs, rem)], out_hbm.at[h, c, pl.ds(rs, rem)])

    pltpu.sync_copy(agg_hbm.at[c, pl.ds(s * RPT, RPT)], tabA.at[pl.ds(s * RPT, RPT)])
    zero_acc(tabB)
    plsc.subcore_barrier()

    for h in range(N_HOPS):
        table, acc = (tabA, tabB) if h % 2 == 0 else (tabB, tabA)
        run_hop(table, acc)
        plsc.subcore_barrier()
        flush(acc, h)
        if h + 1 < N_HOPS:
            zero_acc(table)
            plsc.subcore_barrier()


_hop = pl.kernel(
    _hop_body,
    out_type=jax.ShapeDtypeStruct((N_HOPS, NC, NP, DH), jnp.float32),
    mesh=plsc.VectorSubcoreMesh(core_axis_name="c", subcore_axis_name="s"),
    scratch_types=[
        pltpu.VMEM_SHARED((NP, DH), jnp.float32),
        pltpu.VMEM_SHARED((NP, DH), jnp.float32),
        pltpu.VMEM((4, CHUNK), jnp.int32),
        pltpu.VMEM((4, CHUNK), jnp.int32),
        pltpu.VMEM((4, CHUNK), jnp.float32),
        pltpu.VMEM((2, CHUNK, DH), jnp.float32),
        pltpu.SemaphoreType.DMA((4,)),
        pltpu.SemaphoreType.DMA((2,)),
    ],
)


@jax.jit
def kernel(embed, edge_index, trend):
    row = edge_index[0].astype(jnp.int32)
    col = edge_index[1].astype(jnp.int32)
    pad = EP - N_EDGES
    row = jnp.concatenate([row, jnp.zeros((pad,), jnp.int32)])
    col = jnp.concatenate([col, jnp.zeros((pad,), jnp.int32)])
    tr = jnp.concatenate([trend, jnp.zeros((pad,), jnp.float32)])

    npad = NP - N_NODES
    agg2 = jnp.stack([
        jnp.concatenate([embed[:, :DH], jnp.zeros((npad, DH), jnp.float32)]),
        jnp.concatenate([embed[:, DH:], jnp.zeros((npad, DH), jnp.float32)]),
    ])
    out = _hop(agg2, row, col, tr)
    embs = [embed]
    for h in range(N_HOPS):
        embs.append(jnp.concatenate([out[h, 0, :N_NODES], out[h, 1, :N_NODES]], axis=1))
    return jnp.stack(embs, axis=1)
